# Initial kernel scaffold; baseline (speedup 1.0000x reference)
import jax
import jax.numpy as jnp
from jax.experimental import pallas as pl


def kernel(hidden_states, gate_weight, w13_stacked, w2_stacked):
    raise NotImplementedError("write your pallas kernel here")


# speedup vs baseline: 13.3010x; 13.3010x over previous
import functools

import jax
import jax.numpy as jnp
from jax.experimental import pallas as pl
from jax.experimental.pallas import tpu as pltpu

TOPK = 8
TM = 256


def _ffn_body(tile_ref, exp_ref, start_ref, end_ref,
              x_ref, w13_ref, w2_ref, wrow_ref, y_ref, *, inter):
    g = pl.program_id(0)
    first = jnp.logical_or(g == 0, tile_ref[g] != tile_ref[jnp.maximum(g - 1, 0)])

    xb = x_ref[...]
    w1 = w13_ref[0, :inter, :].astype(jnp.bfloat16)
    w3 = w13_ref[0, inter:, :].astype(jnp.bfloat16)
    w2 = w2_ref[0].astype(jnp.bfloat16)

    dn = (((1,), (1,)), ((), ()))
    a = jax.lax.dot_general(xb, w1, dn, preferred_element_type=jnp.float32)
    b = jax.lax.dot_general(xb, w3, dn, preferred_element_type=jnp.float32)
    h = (a * jax.nn.sigmoid(a) * b).astype(jnp.bfloat16)
    y = jax.lax.dot_general(h, w2, dn, preferred_element_type=jnp.float32)
    y = y * wrow_ref[...]

    iot = jax.lax.broadcasted_iota(jnp.int32, (TM, 1), 0)
    mask = jnp.logical_and(iot >= start_ref[g], iot < end_ref[g])
    yw = y.astype(jnp.bfloat16)

    @pl.when(first)
    def _():
        y_ref[...] = jnp.where(mask, yw, jnp.zeros_like(yw))

    @pl.when(jnp.logical_not(first))
    def _():
        y_ref[...] = jnp.where(mask, yw, y_ref[...])


def _grouped_ffn(x_sorted, w13, w2, wrow, tile_g, exp_g, start_g, end_g):
    P, H = x_sorted.shape
    E, two_i, _ = w13.shape
    inter = two_i // 2
    G = tile_g.shape[0]

    grid_spec = pltpu.PrefetchScalarGridSpec(
        num_scalar_prefetch=4,
        grid=(G,),
        in_specs=[
            pl.BlockSpec((TM, H), lambda g, t, e, s, en: (t[g], 0)),
            pl.BlockSpec((1, two_i, H), lambda g, t, e, s, en: (e[g], 0, 0)),
            pl.BlockSpec((1, H, inter), lambda g, t, e, s, en: (e[g], 0, 0)),
            pl.BlockSpec((TM, 1), lambda g, t, e, s, en: (t[g], 0)),
        ],
        out_specs=pl.BlockSpec((TM, H), lambda g, t, e, s, en: (t[g], 0)),
    )
    return pl.pallas_call(
        functools.partial(_ffn_body, inter=inter),
        grid_spec=grid_spec,
        out_shape=jax.ShapeDtypeStruct((P, H), jnp.bfloat16),
        compiler_params=pltpu.CompilerParams(
            dimension_semantics=("arbitrary",)),
    )(tile_g, exp_g, start_g, end_g, x_sorted, w13, w2, wrow)


def kernel(hidden_states, gate_weight, w13_stacked, w2_stacked):
    orig_shape = hidden_states.shape
    H = orig_shape[-1]
    x = hidden_states.reshape(-1, H)
    M = x.shape[0]
    E = gate_weight.shape[0]
    K = TOPK
    P = M * K
    T = P // TM
    G = T + E - 1

    logits = x @ gate_weight.T
    probs = jax.nn.softmax(logits.astype(jnp.float32), axis=1)
    topk_w, topk_ids = jax.lax.top_k(probs, K)
    topk_w = topk_w / jnp.sum(topk_w, axis=-1, keepdims=True)
    x_bf16 = x.astype(jnp.bfloat16)

    flat_ids = topk_ids.reshape(-1).astype(jnp.int32)
    order = jnp.argsort(flat_ids).astype(jnp.int32)
    token_sorted = order // K
    pos = jnp.zeros((P,), jnp.int32).at[order].set(
        jnp.arange(P, dtype=jnp.int32))
    counts = jnp.bincount(flat_ids, length=E)
    off = jnp.concatenate([jnp.zeros((1,), counts.dtype),
                           jnp.cumsum(counts)]).astype(jnp.int32)

    t_lo = off[:-1] // TM
    t_hi = (off[1:] - 1) // TM
    tt = jnp.arange(T, dtype=jnp.int32)[:, None]
    visits = ((tt >= t_lo[None, :]) & (tt <= t_hi[None, :])
              & (counts[None, :] > 0))
    flat_v = visits.reshape(-1)
    ordv = jnp.argsort(jnp.logical_not(flat_v), stable=True)[:G]
    nvalid = jnp.sum(flat_v.astype(jnp.int32))
    valid = jnp.arange(G, dtype=jnp.int32) < nvalid
    tile_g = jnp.where(valid, ordv // E, T - 1).astype(jnp.int32)
    exp_g = jnp.where(valid, ordv % E, E - 1).astype(jnp.int32)
    start_g = jnp.where(
        valid, jnp.clip(off[exp_g] - tile_g * TM, 0, TM), 0).astype(jnp.int32)
    end_g = jnp.where(
        valid, jnp.clip(off[exp_g + 1] - tile_g * TM, 0, TM), 0).astype(jnp.int32)

    wrow = topk_w.reshape(-1)[order][:, None]

    x_sorted = x_bf16[token_sorted]

    y_sorted = _grouped_ffn(x_sorted, w13_stacked, w2_stacked, wrow,
                            tile_g, exp_g, start_g, end_g)

    y_flat = y_sorted[pos].reshape(M, K, H)
    out = jnp.sum(y_flat.astype(jnp.float32), axis=1)
    return out.reshape(orig_shape).astype(hidden_states.dtype)
